# baseline (speedup 1.0000x reference)
import functools

import jax
import jax.numpy as jnp
from jax import lax
from jax.experimental import pallas as pl
from jax.experimental.pallas import tpu as pltpu
from jax.experimental.pallas import tpu_sc as plsc

LANES = 16
NUM_CORES = 2
NUM_SUBCORES = 16
NUM_WORKERS = NUM_CORES * NUM_SUBCORES


def _make_embed(batch: int, seq: int, hidden: int, vocab: int):
    assert hidden % NUM_WORKERS == 0
    fpw = hidden // NUM_WORKERS
    assert seq % LANES == 0
    tail = max(vocab % 128, 1)

    mesh = plsc.VectorSubcoreMesh(core_axis_name="c", subcore_axis_name="s")

    @functools.partial(
        pl.kernel,
        mesh=mesh,
        compiler_params=pltpu.CompilerParams(needs_layout_passes=False),
        out_type=jax.ShapeDtypeStruct((batch, hidden, seq), jnp.float32),
        scratch_types=[
            pltpu.VMEM((batch, seq), jnp.int32),
            pltpu.VMEM((vocab,), jnp.float32),
            pltpu.VMEM((seq,), jnp.float32),
            pltpu.VMEM((batch, seq), jnp.float32),
            pltpu.VMEM((8, tail), jnp.float32),
            pltpu.SemaphoreType.DMA,
            pltpu.SemaphoreType.DMA,
            pltpu.SemaphoreType.DMA,
        ],
    )
    def embed(x_hbm, tbl_hbm, pos_hbm, out_hbm, x_v, row_v, pos_v, seg_v,
              tail_v, row_sem, pos_sem, out_sem):
        wid = lax.axis_index("s") * NUM_CORES + lax.axis_index("c")
        n_chunk = 8
        vtail = vocab % 128
        vocab_al = vocab - vtail
        chunk = (vocab_al // n_chunk // 128) * 128
        bounds = [i * chunk for i in range(n_chunk)] + [vocab_al]
        for j in range(fpw):
            feat = wid * fpw + j
            row_ref = tbl_hbm.at[feat]
            row_cps = [
                pltpu.async_copy(
                    row_ref.at[pl.ds(bounds[i], bounds[i + 1] - bounds[i])],
                    row_v.at[pl.ds(bounds[i], bounds[i + 1] - bounds[i])],
                    row_sem,
                )
                for i in range(n_chunk)
            ]
            if vtail:
                ftile = pl.multiple_of((feat // 8) * 8, 8)
                row_cps.append(
                    pltpu.async_copy(
                        tbl_hbm.at[pl.ds(ftile, 8), pl.ds(vocab_al, vtail)],
                        tail_v,
                        row_sem,
                    )
                )
            pos_cp = pltpu.async_copy(pos_hbm.at[feat], pos_v, pos_sem)
            if j == 0:
                pltpu.sync_copy(x_hbm, x_v)
            if j > 0:
                for b in range(batch):
                    out_cps[b].wait()
            pos_cp.wait()
            for cp in row_cps:
                cp.wait()
            if vtail:
                frow = feat % 8
                for k in range(vtail // LANES):
                    row_v[pl.ds(vocab_al + k * LANES, LANES)] = tail_v[
                        frow, pl.ds(k * LANES, LANES)
                    ]
            @plsc.parallel_loop(0, seq // LANES, unroll=16)
            def _(k):
                sl = pl.ds(k * LANES, LANES)
                posv = pos_v[sl]
                for b in range(batch):
                    vals = plsc.load_gather(row_v, [x_v[b, sl]])
                    seg_v[b, sl] = vals + posv
            out_cps = [
                pltpu.async_copy(seg_v.at[b], out_hbm.at[b, feat], out_sem)
                for b in range(batch)
            ]
        for b in range(batch):
            out_cps[b].wait()

    return embed


def kernel(x, token_embedding, position_embedding):
    batch, seq = x.shape
    vocab, hidden = token_embedding.shape
    fn = _make_embed(batch, seq, hidden, vocab)
    out = fn(x.astype(jnp.int32), token_embedding.T, position_embedding.T)
    return out.transpose(0, 2, 1)


# speedup vs baseline: 1.0114x; 1.0114x over previous
import functools

import jax
import jax.numpy as jnp
from jax import lax
from jax.experimental import pallas as pl
from jax.experimental.pallas import tpu as pltpu
from jax.experimental.pallas import tpu_sc as plsc

LANES = 16
NUM_CORES = 2
NUM_SUBCORES = 16
NUM_WORKERS = NUM_CORES * NUM_SUBCORES


def _make_embed(batch: int, seq: int, hidden: int, vocab: int):
    assert hidden % NUM_WORKERS == 0
    fpw = hidden // NUM_WORKERS
    assert seq % LANES == 0
    tail = max(vocab % 128, 1)

    mesh = plsc.VectorSubcoreMesh(core_axis_name="c", subcore_axis_name="s")

    @functools.partial(
        pl.kernel,
        mesh=mesh,
        compiler_params=pltpu.CompilerParams(needs_layout_passes=False),
        out_type=jax.ShapeDtypeStruct((batch, hidden, seq), jnp.float32),
        scratch_types=[
            pltpu.VMEM((batch, seq), jnp.int32),
            pltpu.VMEM((vocab,), jnp.float32),
            pltpu.VMEM((seq,), jnp.float32),
            pltpu.VMEM((batch, seq), jnp.float32),
            pltpu.VMEM((8, tail), jnp.float32),
            pltpu.SemaphoreType.DMA,
            pltpu.SemaphoreType.DMA,
            pltpu.SemaphoreType.DMA,
        ],
    )
    def embed(x_hbm, tbl_hbm, pos_hbm, out_hbm, x_v, row_v, pos_v, seg_v,
              tail_v, row_sem, pos_sem, out_sem):
        wid = lax.axis_index("s") * NUM_CORES + lax.axis_index("c")
        n_chunk = 8
        vtail = vocab % 128
        vocab_al = vocab - vtail
        chunk = (vocab_al // n_chunk // 128) * 128
        bounds = [i * chunk for i in range(n_chunk)] + [vocab_al]
        for j in range(fpw):
            feat = wid * fpw + j
            row_ref = tbl_hbm.at[feat]
            row_cps = [
                pltpu.async_copy(
                    row_ref.at[pl.ds(bounds[i], bounds[i + 1] - bounds[i])],
                    row_v.at[pl.ds(bounds[i], bounds[i + 1] - bounds[i])],
                    row_sem,
                )
                for i in range(n_chunk)
            ]
            if vtail:
                ftile = pl.multiple_of((feat // 8) * 8, 8)
                row_cps.append(
                    pltpu.async_copy(
                        tbl_hbm.at[pl.ds(ftile, 8), pl.ds(vocab_al, vtail)],
                        tail_v,
                        row_sem,
                    )
                )
            pos_cp = pltpu.async_copy(pos_hbm.at[feat], pos_v, pos_sem)
            if j == 0:
                pltpu.sync_copy(x_hbm, x_v)
            if j > 0:
                for b in range(batch):
                    out_cps[b].wait()
            pos_cp.wait()
            for cp in row_cps:
                cp.wait()
            if vtail:
                frow = feat % 8
                for k in range(vtail // LANES):
                    row_v[pl.ds(vocab_al + k * LANES, LANES)] = tail_v[
                        frow, pl.ds(k * LANES, LANES)
                    ]
            @plsc.parallel_loop(0, seq // LANES, unroll=8)
            def _(k):
                sl = pl.ds(k * LANES, LANES)
                posv = pos_v[sl]
                for b in range(batch):
                    vals = plsc.load_gather(row_v, [x_v[b, sl]])
                    seg_v[b, sl] = vals + posv
            out_cps = [
                pltpu.async_copy(seg_v.at[b], out_hbm.at[b, feat], out_sem)
                for b in range(batch)
            ]
        for b in range(batch):
            out_cps[b].wait()

    return embed


def kernel(x, token_embedding, position_embedding):
    batch, seq = x.shape
    vocab, hidden = token_embedding.shape
    fn = _make_embed(batch, seq, hidden, vocab)
    out = fn(x.astype(jnp.int32), token_embedding.T, position_embedding.T)
    return out.transpose(0, 2, 1)
